# baseline (speedup 1.0000x reference)
import functools

import jax
import jax.numpy as jnp
from jax.experimental import pallas as pl

N = 10000
E = 320000
D_NODE = 128
D_EDGE = 4
HID = 64
HEADS = 4

NB = 256
EB = 1024
NPAD = 10240
EF = 2 * E + N
EPAD = 650240


def _mm_kernel(x_ref, w_ref, o_ref):
    o_ref[...] = jnp.dot(x_ref[...], w_ref[...],
                         preferred_element_type=jnp.float32)


def _mm(x, w, bm):
    m, k = x.shape
    n = w.shape[1]
    return pl.pallas_call(
        _mm_kernel,
        grid=(m // bm,),
        in_specs=[pl.BlockSpec((bm, k), lambda i: (i, 0)),
                  pl.BlockSpec((k, n), lambda i: (0, 0))],
        out_specs=pl.BlockSpec((bm, n), lambda i: (i, 0)),
        out_shape=jax.ShapeDtypeStruct((m, n), jnp.float32),
    )(x, w)


def _alpha_kernel(a_ref, b_ref, c_ref, o_ref):
    s = a_ref[...] + b_ref[...] + c_ref[...]
    o_ref[...] = jnp.where(s > 0, s, 0.2 * s)


def _edge_alpha(a, b, c):
    e, h = a.shape
    return pl.pallas_call(
        _alpha_kernel,
        grid=(e // EB,),
        in_specs=[pl.BlockSpec((EB, h), lambda i: (i, 0))] * 3,
        out_specs=pl.BlockSpec((EB, h), lambda i: (i, 0)),
        out_shape=jax.ShapeDtypeStruct((e, h), jnp.float32),
    )(a, b, c)


def _ex_kernel(al_ref, am_ref, o_ref):
    am = am_ref[...]
    am = jnp.where(am <= -1e29, 0.0, am)
    o_ref[...] = jnp.exp(al_ref[...] - am)


def _edge_ex(alpha, amax_dst):
    e, h = alpha.shape
    return pl.pallas_call(
        _ex_kernel,
        grid=(e // EB,),
        in_specs=[pl.BlockSpec((EB, h), lambda i: (i, 0))] * 2,
        out_specs=pl.BlockSpec((EB, h), lambda i: (i, 0)),
        out_shape=jax.ShapeDtypeStruct((e, h), jnp.float32),
    )(alpha, amax_dst)


def _msg_kernel(ex_ref, den_ref, xl_ref, k_ref, o_ref):
    t = ex_ref[...] / (den_ref[...] + 1e-16)
    texp = jnp.dot(t, k_ref[...],
                   preferred_element_type=jnp.float32)
    o_ref[...] = xl_ref[...] * texp


def _edge_msg(ex, den_dst, xl_src, c):
    e, h = ex.shape
    hc = xl_src.shape[1]
    row = jax.lax.broadcasted_iota(jnp.int32, (h, hc), 0)
    col = jax.lax.broadcasted_iota(jnp.int32, (h, hc), 1)
    kmat = (col // c == row).astype(jnp.float32)
    return pl.pallas_call(
        _msg_kernel,
        grid=(e // EB,),
        in_specs=[pl.BlockSpec((EB, h), lambda i: (i, 0)),
                  pl.BlockSpec((EB, h), lambda i: (i, 0)),
                  pl.BlockSpec((EB, hc), lambda i: (i, 0)),
                  pl.BlockSpec((h, hc), lambda i: (0, 0))],
        out_specs=pl.BlockSpec((EB, hc), lambda i: (i, 0)),
        out_shape=jax.ShapeDtypeStruct((e, hc), jnp.float32),
    )(ex, den_dst, xl_src, kmat)


def _segmax_kernel(dst_ref, at_ref, o_ref, *, h):
    j = pl.program_id(1)

    @pl.when(j == 0)
    def _():
        o_ref[...] = jnp.full_like(o_ref[...], -1e30)

    i = pl.program_id(0)
    ids = i * NB + jax.lax.broadcasted_iota(jnp.int32, (NB, 1), 0)
    m = dst_ref[0] == ids
    at = at_ref[0]
    cols = []
    for k in range(h):
        cand = jnp.where(m, at[k:k + 1, :], -1e30)
        cols.append(cand.max(axis=1, keepdims=True))
    out = cols[0] if h == 1 else jnp.concatenate(cols, axis=1)
    o_ref[...] = jnp.maximum(o_ref[...], out)


def _segmax(dst3d, alpha_t):
    neb, h, eb = alpha_t.shape
    return pl.pallas_call(
        functools.partial(_segmax_kernel, h=h),
        grid=(NPAD // NB, neb),
        in_specs=[pl.BlockSpec((1, 1, EB), lambda i, j: (j, 0, 0)),
                  pl.BlockSpec((1, h, EB), lambda i, j: (j, 0, 0))],
        out_specs=pl.BlockSpec((NB, h), lambda i, j: (i, 0)),
        out_shape=jax.ShapeDtypeStruct((NPAD, h), jnp.float32),
    )(dst3d, alpha_t)


def _segsum_kernel(dst_ref, v_ref, o_ref):
    j = pl.program_id(1)

    @pl.when(j == 0)
    def _():
        o_ref[...] = jnp.zeros_like(o_ref[...])

    i = pl.program_id(0)
    ids = i * NB + jax.lax.broadcasted_iota(jnp.int32, (NB, 1), 0)
    m = (dst_ref[0] == ids).astype(jnp.float32)
    o_ref[...] += jnp.dot(m, v_ref[...], preferred_element_type=jnp.float32)


def _segsum(dst3d, vals):
    e, k = vals.shape
    return pl.pallas_call(
        _segsum_kernel,
        grid=(NPAD // NB, e // EB),
        in_specs=[pl.BlockSpec((1, 1, EB), lambda i, j: (j, 0, 0)),
                  pl.BlockSpec((EB, k), lambda i, j: (j, 0))],
        out_specs=pl.BlockSpec((NB, k), lambda i, j: (i, 0)),
        out_shape=jax.ShapeDtypeStruct((NPAD, k), jnp.float32),
    )(dst3d, vals)


def _bias_elu_kernel(x_ref, b_ref, o_ref):
    y = x_ref[...] + b_ref[...]
    o_ref[...] = jnp.where(y > 0, y, jnp.exp(jnp.minimum(y, 0.0)) - 1.0)


def _bias_elu(x, b):
    m, k = x.shape
    return pl.pallas_call(
        _bias_elu_kernel,
        grid=(m // NB,),
        in_specs=[pl.BlockSpec((NB, k), lambda i: (i, 0)),
                  pl.BlockSpec((1, k), lambda i: (0, 0))],
        out_specs=pl.BlockSpec((NB, k), lambda i: (i, 0)),
        out_shape=jax.ShapeDtypeStruct((m, k), jnp.float32),
    )(x, b.reshape(1, k))


def _clf_kernel(z_ref, w1_ref, b1_ref, w2_ref, b2_ref, o_ref):
    t = jnp.dot(z_ref[...], w1_ref[...],
                preferred_element_type=jnp.float32) + b1_ref[...]
    t = jnp.maximum(t, 0.0)
    o_ref[...] = jnp.dot(t, w2_ref[...],
                         preferred_element_type=jnp.float32) + b2_ref[...]


def _classifier(z, w1, b1, w2, b2):
    m, k = z.shape
    h = w1.shape[1]
    bm = 512
    return pl.pallas_call(
        _clf_kernel,
        grid=(m // bm,),
        in_specs=[pl.BlockSpec((bm, k), lambda i: (i, 0)),
                  pl.BlockSpec((k, h), lambda i: (0, 0)),
                  pl.BlockSpec((1, h), lambda i: (0, 0)),
                  pl.BlockSpec((h, 1), lambda i: (0, 0)),
                  pl.BlockSpec((1, 1), lambda i: (0, 0))],
        out_specs=pl.BlockSpec((bm, 1), lambda i: (i, 0)),
        out_shape=jax.ShapeDtypeStruct((m, 1), jnp.float32),
    )(z, w1, b1.reshape(1, h), w2, b2.reshape(1, 1))


def _att_vec(w, att, heads, out_c):
    return jnp.einsum('khc,hc->kh', w.reshape(w.shape[0], heads, out_c), att)


def _gat_layer(xp, src, dst, dst3d, ae, w, att_src, att_dst, b, heads, out_c):
    xl = _mm(xp, w, NB)
    als = _mm(xp, _att_vec(w, att_src, heads, out_c), NB)
    ald = _mm(xp, _att_vec(w, att_dst, heads, out_c), NB)
    alpha = _edge_alpha(jnp.take(als, src, axis=0),
                        jnp.take(ald, dst, axis=0), ae)
    alpha_t = alpha.reshape(EPAD // EB, EB, heads).transpose(0, 2, 1)
    amax = _segmax(dst3d, alpha_t)
    ex = _edge_ex(alpha, jnp.take(amax, dst, axis=0))
    den = _segsum(dst3d, ex)
    msg = _edge_msg(ex, jnp.take(den, dst, axis=0),
                    jnp.take(xl, src, axis=0), out_c)
    out = _segsum(dst3d, msg)
    return _bias_elu(out, b)


def kernel(x, edge_index, edge_attr, pairs, W1, att_src1, att_dst1, We1,
           att_e1, b1, W2, att_src2, att_dst2, We2, att_e2, b2,
           Wc1, bc1, Wc2, bc2):
    src0, dst0 = edge_index[0], edge_index[1]
    loop = jnp.arange(N, dtype=src0.dtype)
    src = jnp.concatenate([src0, dst0, loop])
    dst = jnp.concatenate([dst0, src0, loop])
    ea_loop = jnp.broadcast_to(edge_attr.mean(axis=0, keepdims=True),
                               (N, D_EDGE))
    ea = jnp.concatenate([edge_attr, edge_attr, ea_loop], axis=0)

    src = jnp.pad(src, (0, EPAD - EF))
    dst = jnp.pad(dst, (0, EPAD - EF), constant_values=NPAD)
    ea = jnp.pad(ea, ((0, EPAD - EF), (0, 0)))
    dst3d = dst.reshape(EPAD // EB, 1, EB)
    xp = jnp.pad(x, ((0, NPAD - N), (0, 0)))

    ae1 = _mm(ea, _att_vec(We1, att_e1, HEADS, HID), EB)
    h1 = _gat_layer(xp, src, dst, dst3d, ae1, W1, att_src1, att_dst1, b1,
                    HEADS, HID)
    ae2 = _mm(ea, _att_vec(We2, att_e2, 1, HID), EB)
    h2 = _gat_layer(h1, src, dst, dst3d, ae2, W2, att_src2, att_dst2, b2,
                    1, HID)

    h = h2[:N]
    z = jnp.concatenate([jnp.take(h, pairs[:, 0], axis=0),
                         jnp.take(h, pairs[:, 1], axis=0)], axis=1)
    logits = _classifier(z, Wc1, bc1, Wc2, bc2)
    return logits[:, 0]


# speedup vs baseline: 1.7022x; 1.7022x over previous
import functools

import jax
import jax.numpy as jnp
from jax.experimental import pallas as pl

N = 10000
E = 320000
D_NODE = 128
D_EDGE = 4
HID = 64
HEADS = 4

NB = 1024
EB = 1024
NPAD = 10240
EF = 2 * E + N
EPAD = 650240


def _mm_kernel(x_ref, w_ref, o_ref):
    o_ref[...] = jnp.dot(x_ref[...], w_ref[...],
                         preferred_element_type=jnp.float32)


def _mm(x, w, bm):
    m, k = x.shape
    n = w.shape[1]
    return pl.pallas_call(
        _mm_kernel,
        grid=(m // bm,),
        in_specs=[pl.BlockSpec((bm, k), lambda i: (i, 0)),
                  pl.BlockSpec((k, n), lambda i: (0, 0))],
        out_specs=pl.BlockSpec((bm, n), lambda i: (i, 0)),
        out_shape=jax.ShapeDtypeStruct((m, n), jnp.float32),
    )(x, w)


def _alpha_kernel(a_ref, b_ref, c_ref, o_ref):
    s = a_ref[...] + b_ref[...] + c_ref[...]
    o_ref[...] = jnp.where(s > 0, s, 0.2 * s)


def _edge_alpha(a, b, c):
    e, h = a.shape
    return pl.pallas_call(
        _alpha_kernel,
        grid=(e // EB,),
        in_specs=[pl.BlockSpec((EB, h), lambda i: (i, 0))] * 3,
        out_specs=pl.BlockSpec((EB, h), lambda i: (i, 0)),
        out_shape=jax.ShapeDtypeStruct((e, h), jnp.float32),
    )(a, b, c)


def _ex_kernel(al_ref, am_ref, o_ref):
    am = am_ref[...]
    am = jnp.where(am <= -1e29, 0.0, am)
    o_ref[...] = jnp.exp(al_ref[...] - am)


def _edge_ex(alpha, amax_dst):
    e, h = alpha.shape
    return pl.pallas_call(
        _ex_kernel,
        grid=(e // EB,),
        in_specs=[pl.BlockSpec((EB, h), lambda i: (i, 0))] * 2,
        out_specs=pl.BlockSpec((EB, h), lambda i: (i, 0)),
        out_shape=jax.ShapeDtypeStruct((e, h), jnp.float32),
    )(alpha, amax_dst)


def _msg_kernel(ex_ref, den_ref, xl_ref, k_ref, o_ref):
    t = ex_ref[...] / (den_ref[...] + 1e-16)
    texp = jnp.dot(t, k_ref[...],
                   preferred_element_type=jnp.float32)
    o_ref[...] = xl_ref[...] * texp


def _edge_msg(ex, den_dst, xl_src, c):
    e, h = ex.shape
    hc = xl_src.shape[1]
    row = jax.lax.broadcasted_iota(jnp.int32, (h, hc), 0)
    col = jax.lax.broadcasted_iota(jnp.int32, (h, hc), 1)
    kmat = (col // c == row).astype(jnp.float32)
    return pl.pallas_call(
        _msg_kernel,
        grid=(e // EB,),
        in_specs=[pl.BlockSpec((EB, h), lambda i: (i, 0)),
                  pl.BlockSpec((EB, h), lambda i: (i, 0)),
                  pl.BlockSpec((EB, hc), lambda i: (i, 0)),
                  pl.BlockSpec((h, hc), lambda i: (0, 0))],
        out_specs=pl.BlockSpec((EB, hc), lambda i: (i, 0)),
        out_shape=jax.ShapeDtypeStruct((e, hc), jnp.float32),
    )(ex, den_dst, xl_src, kmat)


def _segmax_kernel(dst_ref, at_ref, o_ref, *, h):
    j = pl.program_id(1)

    @pl.when(j == 0)
    def _():
        o_ref[...] = jnp.full_like(o_ref[...], -1e30)

    i = pl.program_id(0)
    ids = i * NB + jax.lax.broadcasted_iota(jnp.int32, (NB, 1), 0)
    m = dst_ref[0] == ids
    at = at_ref[0]
    cols = []
    for k in range(h):
        cand = jnp.where(m, at[k:k + 1, :], -1e30)
        cols.append(cand.max(axis=1, keepdims=True))
    out = cols[0] if h == 1 else jnp.concatenate(cols, axis=1)
    o_ref[...] = jnp.maximum(o_ref[...], out)


def _segmax(dst3d, alpha_t):
    neb, h, eb = alpha_t.shape
    return pl.pallas_call(
        functools.partial(_segmax_kernel, h=h),
        grid=(NPAD // NB, neb),
        in_specs=[pl.BlockSpec((1, 1, EB), lambda i, j: (j, 0, 0)),
                  pl.BlockSpec((1, h, EB), lambda i, j: (j, 0, 0))],
        out_specs=pl.BlockSpec((NB, h), lambda i, j: (i, 0)),
        out_shape=jax.ShapeDtypeStruct((NPAD, h), jnp.float32),
    )(dst3d, alpha_t)


def _segsum_kernel(dst_ref, v_ref, o_ref):
    j = pl.program_id(1)

    @pl.when(j == 0)
    def _():
        o_ref[...] = jnp.zeros_like(o_ref[...])

    i = pl.program_id(0)
    ids = i * NB + jax.lax.broadcasted_iota(jnp.int32, (NB, 1), 0)
    m = (dst_ref[0] == ids).astype(jnp.float32)
    o_ref[...] += jnp.dot(m, v_ref[...], preferred_element_type=jnp.float32)


def _segsum(dst3d, vals):
    e, k = vals.shape
    return pl.pallas_call(
        _segsum_kernel,
        grid=(NPAD // NB, e // EB),
        in_specs=[pl.BlockSpec((1, 1, EB), lambda i, j: (j, 0, 0)),
                  pl.BlockSpec((EB, k), lambda i, j: (j, 0))],
        out_specs=pl.BlockSpec((NB, k), lambda i, j: (i, 0)),
        out_shape=jax.ShapeDtypeStruct((NPAD, k), jnp.float32),
    )(dst3d, vals)


def _bias_elu_kernel(x_ref, b_ref, o_ref):
    y = x_ref[...] + b_ref[...]
    o_ref[...] = jnp.where(y > 0, y, jnp.exp(jnp.minimum(y, 0.0)) - 1.0)


def _bias_elu(x, b):
    m, k = x.shape
    return pl.pallas_call(
        _bias_elu_kernel,
        grid=(m // NB,),
        in_specs=[pl.BlockSpec((NB, k), lambda i: (i, 0)),
                  pl.BlockSpec((1, k), lambda i: (0, 0))],
        out_specs=pl.BlockSpec((NB, k), lambda i: (i, 0)),
        out_shape=jax.ShapeDtypeStruct((m, k), jnp.float32),
    )(x, b.reshape(1, k))


def _clf_kernel(z_ref, w1_ref, b1_ref, w2_ref, b2_ref, o_ref):
    t = jnp.dot(z_ref[...], w1_ref[...],
                preferred_element_type=jnp.float32) + b1_ref[...]
    t = jnp.maximum(t, 0.0)
    o_ref[...] = jnp.dot(t, w2_ref[...],
                         preferred_element_type=jnp.float32) + b2_ref[...]


def _classifier(z, w1, b1, w2, b2):
    m, k = z.shape
    h = w1.shape[1]
    bm = 512
    return pl.pallas_call(
        _clf_kernel,
        grid=(m // bm,),
        in_specs=[pl.BlockSpec((bm, k), lambda i: (i, 0)),
                  pl.BlockSpec((k, h), lambda i: (0, 0)),
                  pl.BlockSpec((1, h), lambda i: (0, 0)),
                  pl.BlockSpec((h, 1), lambda i: (0, 0)),
                  pl.BlockSpec((1, 1), lambda i: (0, 0))],
        out_specs=pl.BlockSpec((bm, 1), lambda i: (i, 0)),
        out_shape=jax.ShapeDtypeStruct((m, 1), jnp.float32),
    )(z, w1, b1.reshape(1, h), w2, b2.reshape(1, 1))


def _att_vec(w, att, heads, out_c):
    return jnp.einsum('khc,hc->kh', w.reshape(w.shape[0], heads, out_c), att)


def _gat_layer(xp, src, dst, dst3d, ae, w, att_src, att_dst, b, heads, out_c):
    xl = _mm(xp, w, NB)
    als = _mm(xp, _att_vec(w, att_src, heads, out_c), NB)
    ald = _mm(xp, _att_vec(w, att_dst, heads, out_c), NB)
    alpha = _edge_alpha(jnp.take(als, src, axis=0),
                        jnp.take(ald, dst, axis=0), ae)
    alpha_t = alpha.reshape(EPAD // EB, EB, heads).transpose(0, 2, 1)
    amax = _segmax(dst3d, alpha_t)
    ex = _edge_ex(alpha, jnp.take(amax, dst, axis=0))
    den = _segsum(dst3d, ex)
    msg = _edge_msg(ex, jnp.take(den, dst, axis=0),
                    jnp.take(xl, src, axis=0), out_c)
    out = _segsum(dst3d, msg)
    return _bias_elu(out, b)


def kernel(x, edge_index, edge_attr, pairs, W1, att_src1, att_dst1, We1,
           att_e1, b1, W2, att_src2, att_dst2, We2, att_e2, b2,
           Wc1, bc1, Wc2, bc2):
    src0, dst0 = edge_index[0], edge_index[1]
    loop = jnp.arange(N, dtype=src0.dtype)
    src = jnp.concatenate([src0, dst0, loop])
    dst = jnp.concatenate([dst0, src0, loop])
    ea_loop = jnp.broadcast_to(edge_attr.mean(axis=0, keepdims=True),
                               (N, D_EDGE))
    ea = jnp.concatenate([edge_attr, edge_attr, ea_loop], axis=0)

    src = jnp.pad(src, (0, EPAD - EF))
    dst = jnp.pad(dst, (0, EPAD - EF), constant_values=NPAD)
    ea = jnp.pad(ea, ((0, EPAD - EF), (0, 0)))
    dst3d = dst.reshape(EPAD // EB, 1, EB)
    xp = jnp.pad(x, ((0, NPAD - N), (0, 0)))

    ae1 = _mm(ea, _att_vec(We1, att_e1, HEADS, HID), EB)
    h1 = _gat_layer(xp, src, dst, dst3d, ae1, W1, att_src1, att_dst1, b1,
                    HEADS, HID)
    ae2 = _mm(ea, _att_vec(We2, att_e2, 1, HID), EB)
    h2 = _gat_layer(h1, src, dst, dst3d, ae2, W2, att_src2, att_dst2, b2,
                    1, HID)

    h = h2[:N]
    z = jnp.concatenate([jnp.take(h, pairs[:, 0], axis=0),
                         jnp.take(h, pairs[:, 1], axis=0)], axis=1)
    logits = _classifier(z, Wc1, bc1, Wc2, bc2)
    return logits[:, 0]
